# baseline (speedup 1.0000x reference)
import functools

import jax
import jax.numpy as jnp
from jax import lax
from jax.experimental import pallas as pl
from jax.experimental.pallas import tpu as pltpu
from jax.experimental.pallas import tpu_sc as plsc

_NU = 6000
_NI = 4000
_N = _NU + _NI
_E = 160000
_D = 256
_H = 128
_NS = 16
_EB = 64
_NBLK = 160
_EPT = _EB * _NBLK
_EPAD = _EPT * _NS
_NP = 10240
_STR = _NP // _NS
_ZR = _EB
_NB = 4
_GS = 4


def _sc_gcn(x0_lo, x0_hi, rows, cols, vals, zchunk, interpret=False):
    mesh = plsc.VectorSubcoreMesh(core_axis_name="c", subcore_axis_name="s",
                                  num_cores=2, num_subcores=_NS)
    out_t = [jax.ShapeDtypeStruct((_NP, _H), jnp.float32) for _ in range(6)]

    @functools.partial(
        pl.kernel,
        out_type=out_t,
        mesh=mesh,
        interpret=interpret,
        compiler_params=pltpu.CompilerParams(needs_layout_passes=False),
        scratch_types=(
            [pltpu.VMEM((_NBLK // 2, 2 * _EB), jnp.int32)]
            + [pltpu.VMEM((_EB,), jnp.int32)] * _NB
            + [pltpu.VMEM((_EB,), jnp.float32)] * _NB
            + [pltpu.VMEM((_EB, _H), jnp.float32)] * _NB
            + [pltpu.VMEM_SHARED((_NP, _H), jnp.float32)]
            + [pltpu.SemaphoreType.DMA] * (4 * _NB)
        ),
    )
    def k(x0l, x0h, rws, cls, vls, zz,
          o1l, o2l, o3l, o1h, o2h, o3h,
          cv, *rest):
        rvs = rest[:_NB]
        vvs = rest[_NB:2 * _NB]
        gbs = rest[2 * _NB:3 * _NB]
        acc = rest[3 * _NB]
        sems = rest[3 * _NB + 1:]
        sgs = sems[:_NB]
        sss = sems[_NB:2 * _NB]
        srs = sems[2 * _NB:3 * _NB]
        svs = sems[3 * _NB:4 * _NB]
        c = lax.axis_index("c")
        s = lax.axis_index("s")
        pltpu.sync_copy(cls.at[s], cv)

        def scale(gb, vvk):
            def scale16(jb, carry2):
                v_all = vvk[pl.ds(jb * 16, 16)]
                for t in range(16):
                    v16 = jnp.take(
                        v_all, jnp.full((16,), t, jnp.int32), mode="wrap")
                    j = jb * 16 + t
                    for cc in range(_H // 16):
                        sl = pl.ds(cc * 16, 16)
                        gb[j, sl] = gb[j, sl] * v16
                return carry2

            lax.fori_loop(0, _EB // 16, scale16, 0)

        def half(src0, outs):
            srcs = [src0, outs[0], outs[1]]
            for l in range(3):
                pltpu.sync_copy(zz, gbs[0])
                for i in range(_STR // _EB):
                    pltpu.sync_copy(
                        gbs[0], acc.at[pl.ds(s * _STR + i * _EB, _EB)])
                plsc.subcore_barrier()
                src = srcs[l]

                def issue(b, b2, off, kk):
                    pltpu.async_copy(rws.at[s, b], rvs[kk], srs[kk])
                    pltpu.async_copy(vls.at[s, b], vvs[kk], svs[kk])
                    sub = _EB // _GS
                    for u in range(_GS):
                        pltpu.async_copy(
                            src.at[cv.at[b2, pl.ds(off + u * sub, sub)]],
                            gbs[kk].at[pl.ds(u * sub, sub)], sgs[kk])

                for kk in range(_NB):
                    issue(kk, kk // 2, (kk % 2) * _EB, kk)

                def body(i, carry):
                    for kk in range(_NB):
                        b = i * _NB + kk
                        b2 = i * 2 + kk // 2
                        off = (kk % 2) * _EB
                        pltpu.make_async_copy(
                            src.at[cv.at[b2, pl.ds(off, _EB)]],
                            gbs[kk], sgs[kk]).wait()
                        pltpu.make_async_copy(
                            vls.at[s, b], vvs[kk], svs[kk]).wait()
                        scale(gbs[kk], vvs[kk])
                        pltpu.make_async_copy(
                            rws.at[s, b], rvs[kk], srs[kk]).wait()
                        pltpu.async_copy(
                            gbs[kk], acc.at[rvs[kk]], sss[kk], add=True)
                        pk = (kk + _NB - 1) % _NB

                        @pl.when(b >= 1)
                        def _():
                            pltpu.make_async_copy(
                                gbs[pk], acc.at[rvs[pk]], sss[pk]).wait()

                            @pl.when(b + _NB - 1 < _NBLK)
                            def _():
                                issue(b + _NB - 1, i * 2 + (kk + 3) // 2,
                                      ((kk + 1) % 2) * _EB, pk)
                    return carry

                lax.fori_loop(0, _NBLK // _NB, body, 0)
                pltpu.make_async_copy(
                    gbs[_NB - 1], acc.at[rvs[_NB - 1]],
                    sss[_NB - 1]).wait()
                plsc.subcore_barrier()
                pltpu.sync_copy(acc.at[pl.ds(s * _STR, _STR)],
                                outs[l].at[pl.ds(s * _STR, _STR)])
                plsc.subcore_barrier()

        pl.when(c == 0)(lambda: half(x0l, [o1l, o2l, o3l]))
        pl.when(c == 1)(lambda: half(x0h, [o1h, o2h, o3h]))

    return k(x0_lo, x0_hi, rows, cols, vals, zchunk)


def _tc_finish(x0_lo, x0_hi, l1l, l2l, l3l, l1h, l2h, l3h,
               normalize, interpret=False):
    rb = 1000
    grid = (_N // rb,)

    def body(al, ah, bl, cl, dl, bh, ch, dh, o):
        lo = (al[...] + bl[...] + cl[...] + dl[...]) * 0.25
        hi = (ah[...] + bh[...] + ch[...] + dh[...]) * 0.25
        if normalize:
            nrm = jnp.sqrt(jnp.sum(lo * lo, axis=1, keepdims=True)
                           + jnp.sum(hi * hi, axis=1, keepdims=True))
            nrm = jnp.maximum(nrm, 1e-12)
            lo = lo / nrm
            hi = hi / nrm
        o[:, :_H] = lo
        o[:, _H:] = hi

    half_spec = pl.BlockSpec((rb, _H), lambda i: (i, 0))
    return pl.pallas_call(
        body,
        grid=grid,
        in_specs=[half_spec] * 8,
        out_specs=pl.BlockSpec((rb, _D), lambda i: (i, 0)),
        out_shape=jax.ShapeDtypeStruct((_N, _D), jnp.float32),
        interpret=interpret,
    )(x0_lo, x0_hi, l1l, l2l, l3l, l1h, l2h, l3h)


def _pad_edges(indices, values):
    rows = indices[0].astype(jnp.int32)
    cols = indices[1].astype(jnp.int32)
    vals = values.astype(jnp.float32)
    pad = _EPAD - _E
    rows = jnp.pad(rows, (0, pad)).reshape(_NS, _NBLK, _EB)
    cols = jnp.pad(cols, (0, pad)).reshape(_NS, _NBLK // 2, 2 * _EB)
    vals = jnp.pad(vals, (0, pad)).reshape(_NS, _NBLK, _EB)
    return rows, cols, vals


def kernel(user_emb, item_emb, adj_indices, adj_values,
           sg1_indices, sg1_values, sg2_indices, sg2_values,
           users, items, neg_items, interpret=False):
    x0 = jnp.concatenate([user_emb, item_emb], axis=0)
    x0_lo = x0[:, :_H]
    x0_hi = x0[:, _H:]
    zchunk = jnp.zeros((_ZR, _H), jnp.float32)

    outs = []
    for (idx, val), normalize in (
            ((adj_indices, adj_values), False),
            ((sg1_indices, sg1_values), True),
            ((sg2_indices, sg2_values), True)):
        rows, cols, vals = _pad_edges(idx, val)
        louts = _sc_gcn(
            x0_lo, x0_hi, rows, cols, vals, zchunk, interpret=interpret)
        l1l, l2l, l3l, l1h, l2h, l3h = (o[:_N] for o in louts)
        m = _tc_finish(x0_lo, x0_hi, l1l, l2l, l3l, l1h, l2h, l3h,
                       normalize, interpret=interpret)
        outs.append(m)

    ma, m1, m2 = outs
    return (ma[:_NU], ma[_NU:], m1[:_NU], m1[_NU:], m2[:_NU], m2[_NU:])
